# Initial kernel scaffold; baseline (speedup 1.0000x reference)
import jax
import jax.numpy as jnp
from jax.experimental import pallas as pl


def kernel(in_values, values, indices, bias):
    raise NotImplementedError("write your pallas kernel here")


# speedup vs baseline: 48.5757x; 48.5757x over previous
import functools

import jax
import jax.numpy as jnp
from jax import lax
from jax.experimental import pallas as pl
from jax.experimental.pallas import tpu as pltpu
from jax.experimental.pallas import tpu_sc as plsc

N_IN = 2048
N_OUT = 2048
N_TILES = 32
HALF_ROWS = 32
HALF_W = HALF_ROWS * N_IN
N_BLKS = N_OUT // HALF_ROWS
SUB = 8192


def _sc_densify_body(bounds_hbm, flat_hbm, vals_hbm, w_hbm,
                     bnd_v, idx_v, val_v, wbuf):
    c = lax.axis_index("c")
    s = lax.axis_index("s")
    wid = s * 2 + c
    pltpu.sync_copy(bounds_hbm, bnd_v)
    zero16 = jnp.zeros((16,), jnp.float32)
    lanes0 = lax.iota(jnp.int32, 16)

    for h in range(2):
        blk = wid * 2 + h
        bvec = bnd_v[pl.ds(blk, 16)]
        b0 = bvec[0]
        b1 = bvec[1]
        base = blk * HALF_W

        def zbody(i, _):
            wbuf[pl.ds(i * 16, 16)] = zero16
            return 0
        lax.fori_loop(0, HALF_W // 16, zbody, 0, unroll=8)

        s0 = b0 & ~7
        off = b0 - s0
        tot = off + (b1 - b0)
        nsub = (tot + SUB - 1) // SUB

        def sub_body(j, _):
            start = pl.multiple_of(s0 + j * SUB, 8)
            pltpu.sync_copy(flat_hbm.at[pl.ds(start, SUB)], idx_v)
            pltpu.sync_copy(vals_hbm.at[pl.ds(start, SUB)], val_v)
            lo = off - j * SUB
            hi = tot - j * SUB

            def scat(i, _):
                lane = lanes0 + i * 16
                m = (lane >= lo) & (lane < hi)
                fi = idx_v[pl.ds(i * 16, 16)] - base
                v = val_v[pl.ds(i * 16, 16)]
                plsc.store_scatter(wbuf, [fi], v, mask=m)
                return 0
            lax.fori_loop(0, SUB // 16, scat, 0, unroll=8)
            return 0
        lax.fori_loop(0, nsub, sub_body, 0)

        pltpu.sync_copy(wbuf, w_hbm.at[pl.ds(base, HALF_W)])


def _densify(bounds, flat, vals):
    mesh = plsc.VectorSubcoreMesh(core_axis_name="c", subcore_axis_name="s")
    return pl.kernel(
        _sc_densify_body,
        out_type=jax.ShapeDtypeStruct((N_OUT * N_IN,), jnp.float32),
        mesh=mesh,
        scratch_types=[
            pltpu.VMEM((128,), jnp.int32),
            pltpu.VMEM((SUB,), jnp.int32),
            pltpu.VMEM((SUB,), jnp.float32),
            pltpu.VMEM((HALF_W,), jnp.float32),
        ],
        compiler_params=pltpu.CompilerParams(needs_layout_passes=False),
    )(bounds, flat, vals)


BM = 1024
BN = 1024


def _mm_body(x_ref, w_ref, b_ref, o_ref):
    xb = x_ref[...].astype(jnp.bfloat16)
    wb = w_ref[...].astype(jnp.bfloat16)
    acc = lax.dot_general(xb, wb, (((1,), (1,)), ((), ())),
                          preferred_element_type=jnp.float32)
    o_ref[...] = acc + b_ref[...]


def _matmul(x, w, bias2):
    batch = x.shape[0]
    return pl.pallas_call(
        _mm_body,
        grid=(N_OUT // BN, batch // BM),
        in_specs=[
            pl.BlockSpec((BM, N_IN), lambda j, i: (i, 0)),
            pl.BlockSpec((BN, N_IN), lambda j, i: (j, 0)),
            pl.BlockSpec((1, BN), lambda j, i: (0, j)),
        ],
        out_specs=pl.BlockSpec((BM, BN), lambda j, i: (i, j)),
        out_shape=jax.ShapeDtypeStruct((batch, N_OUT), jnp.float32),
    )(x, w, bias2)


def kernel(in_values, values, indices, bias):
    rows = indices[0].astype(jnp.int32)
    cols = indices[1].astype(jnp.int32)
    flat = rows * N_IN + cols
    bounds = jnp.searchsorted(
        rows, jnp.arange(0, N_OUT + 1, HALF_ROWS, dtype=jnp.int32)
    ).astype(jnp.int32)
    bounds = jnp.pad(bounds, (0, 128 - bounds.shape[0]))
    flat_p = jnp.pad(flat, (0, SUB + 8))
    vals_p = jnp.pad(values, (0, SUB + 8))

    w_flat = _densify(bounds, flat_p, vals_p)
    w = w_flat.reshape(N_OUT, N_IN)
    out = _matmul(in_values, w, bias.reshape(1, N_OUT))
    return out
